# baseline (device time: 21142 ns/iter reference)
import jax
import jax.numpy as jnp
from jax import lax
from jax.experimental import pallas as pl
from jax.experimental.pallas import tpu as pltpu

N_DEV = 4
EPS = 1e-5
NB = 4


def kernel(x, gamma, beta):
    m, n = x.shape
    n_global = n * N_DEV
    mb = m // NB

    def body(x_hbm, g_ref, b_ref, out_hbm,
             xv_ref, ov_ref, loc_ref, rbuf_ref,
             in_sems, out_sems, send_sems, recv_sems):
        my = lax.axis_index("i")

        in_copies = []
        for k in range(NB):
            cp = pltpu.make_async_copy(
                x_hbm.at[pl.ds(k * mb, mb), :],
                xv_ref.at[pl.ds(k * mb, mb), :],
                in_sems.at[k],
            )
            cp.start()
            in_copies.append(cp)

        ones = jnp.ones((1, n), jnp.float32)
        dims = (((1,), (1,)), ((), ()))

        sends = []
        for k in range(NB):
            in_copies[k].wait()
            xb = xv_ref[k * mb:(k + 1) * mb, :]
            s = lax.dot_general(ones, xb, dims, preferred_element_type=jnp.float32)
            q = lax.dot_general(ones, xb * xb, dims, preferred_element_type=jnp.float32)
            loc_ref[0:1, k * mb:(k + 1) * mb] = s
            loc_ref[1:2, k * mb:(k + 1) * mb] = q
            for off in (1, 2, 3):
                peer = (my + off) % N_DEV
                rdma = pltpu.make_async_remote_copy(
                    src_ref=loc_ref.at[:, pl.ds(k * mb, mb)],
                    dst_ref=rbuf_ref.at[3 - off, :, pl.ds(k * mb, mb)],
                    send_sem=send_sems.at[off - 1, k],
                    recv_sem=recv_sems.at[3 - off, k],
                    device_id=(peer,),
                    device_id_type=pl.DeviceIdType.MESH,
                )
                rdma.start()
                sends.append(rdma)

        out_copies = []
        for k in range(NB):
            blk = pl.ds(k * mb, mb)
            for j in range(3):
                recv = pltpu.make_async_remote_copy(
                    src_ref=loc_ref.at[:, blk],
                    dst_ref=rbuf_ref.at[j, :, blk],
                    send_sem=send_sems.at[0, k],
                    recv_sem=recv_sems.at[j, k],
                    device_id=(my,),
                    device_id_type=pl.DeviceIdType.MESH,
                )
                recv.wait_recv()
            tot = (loc_ref[:, k * mb:(k + 1) * mb]
                   + rbuf_ref[0, :, k * mb:(k + 1) * mb]
                   + rbuf_ref[1, :, k * mb:(k + 1) * mb]
                   + rbuf_ref[2, :, k * mb:(k + 1) * mb])
            mean_r = tot[0:1, :] / n_global
            var_r = tot[1:2, :] / n_global - mean_r * mean_r
            rstd_r = lax.rsqrt(var_r + EPS)
            mv = jnp.concatenate([mean_r, rstd_r], axis=0)
            t = mv.T
            mean_c = t[:, 0:1]
            rstd_c = t[:, 1:2]
            xb = xv_ref[k * mb:(k + 1) * mb, :]
            ov_ref[k * mb:(k + 1) * mb, :] = (
                (xb - mean_c) * rstd_c * g_ref[:, :] + b_ref[:, :]
            )
            cp = pltpu.make_async_copy(
                ov_ref.at[blk, :], out_hbm.at[blk, :], out_sems.at[k]
            )
            cp.start()
            out_copies.append(cp)

        for rdma in sends:
            rdma.wait_send()
        for cp in out_copies:
            cp.wait()

    return pl.pallas_call(
        body,
        out_shape=jax.ShapeDtypeStruct((m, n), jnp.float32),
        in_specs=[
            pl.BlockSpec(memory_space=pltpu.HBM),
            pl.BlockSpec(memory_space=pltpu.VMEM),
            pl.BlockSpec(memory_space=pltpu.VMEM),
        ],
        out_specs=pl.BlockSpec(memory_space=pltpu.HBM),
        scratch_shapes=[
            pltpu.VMEM((m, n), jnp.float32),
            pltpu.VMEM((m, n), jnp.float32),
            pltpu.VMEM((2, m), jnp.float32),
            pltpu.VMEM((3, 2, m), jnp.float32),
            pltpu.SemaphoreType.DMA((NB,)),
            pltpu.SemaphoreType.DMA((NB,)),
            pltpu.SemaphoreType.DMA((3, NB)),
            pltpu.SemaphoreType.DMA((3, NB)),
        ],
    )(x, gamma.reshape(1, n), beta.reshape(1, n))


# device time: 18465 ns/iter; 1.1450x vs baseline; 1.1450x over previous
import jax
import jax.numpy as jnp
from jax import lax
from jax.experimental import pallas as pl
from jax.experimental.pallas import tpu as pltpu

N_DEV = 4
EPS = 1e-5
NB = 4


def kernel(x, gamma, beta):
    m, n = x.shape
    n_global = n * N_DEV
    mb = m // NB

    def body(x_hbm, g_ref, b_ref, out_hbm,
             xv_ref, ov_ref, loc_ref, rbuf_ref,
             in_sems, out_sems, send_sems, recv_sems):
        my = lax.axis_index("i")

        in_copies = []
        for k in range(NB):
            cp = pltpu.make_async_copy(
                x_hbm.at[pl.ds(k * mb, mb), :],
                xv_ref.at[pl.ds(k * mb, mb), :],
                in_sems.at[k],
            )
            cp.start()
            in_copies.append(cp)

        barrier_sem = pltpu.get_barrier_semaphore()
        for off in (1, 2, 3):
            pl.semaphore_signal(
                barrier_sem, inc=1,
                device_id=((my + off) % N_DEV,),
                device_id_type=pl.DeviceIdType.MESH,
            )
        pl.semaphore_wait(barrier_sem, 3)

        ones = jnp.ones((1, n), jnp.float32)
        dims = (((1,), (1,)), ((), ()))

        sends = []
        for k in range(NB):
            in_copies[k].wait()
            xb = xv_ref[k * mb:(k + 1) * mb, :]
            s = lax.dot_general(ones, xb, dims, preferred_element_type=jnp.float32)
            q = lax.dot_general(ones, xb * xb, dims, preferred_element_type=jnp.float32)
            loc_ref[0:1, k * mb:(k + 1) * mb] = s
            loc_ref[1:2, k * mb:(k + 1) * mb] = q
            for off in (1, 2, 3):
                peer = (my + off) % N_DEV
                rdma = pltpu.make_async_remote_copy(
                    src_ref=loc_ref.at[:, pl.ds(k * mb, mb)],
                    dst_ref=rbuf_ref.at[3 - off, :, pl.ds(k * mb, mb)],
                    send_sem=send_sems.at[off - 1, k],
                    recv_sem=recv_sems.at[3 - off, k],
                    device_id=(peer,),
                    device_id_type=pl.DeviceIdType.MESH,
                )
                rdma.start()
                sends.append(rdma)

        out_copies = []
        for k in range(NB):
            blk = pl.ds(k * mb, mb)
            for j in range(3):
                recv = pltpu.make_async_remote_copy(
                    src_ref=loc_ref.at[:, blk],
                    dst_ref=rbuf_ref.at[j, :, blk],
                    send_sem=send_sems.at[0, k],
                    recv_sem=recv_sems.at[j, k],
                    device_id=(my,),
                    device_id_type=pl.DeviceIdType.MESH,
                )
                recv.wait_recv()
            tot = (loc_ref[:, k * mb:(k + 1) * mb]
                   + rbuf_ref[0, :, k * mb:(k + 1) * mb]
                   + rbuf_ref[1, :, k * mb:(k + 1) * mb]
                   + rbuf_ref[2, :, k * mb:(k + 1) * mb])
            mean_r = tot[0:1, :] / n_global
            var_r = tot[1:2, :] / n_global - mean_r * mean_r
            rstd_r = lax.rsqrt(var_r + EPS)
            mv = jnp.concatenate([mean_r, rstd_r], axis=0)
            t = mv.T
            mean_c = t[:, 0:1]
            rstd_c = t[:, 1:2]
            xb = xv_ref[k * mb:(k + 1) * mb, :]
            ov_ref[k * mb:(k + 1) * mb, :] = (
                (xb - mean_c) * rstd_c * g_ref[:, :] + b_ref[:, :]
            )
            cp = pltpu.make_async_copy(
                ov_ref.at[blk, :], out_hbm.at[blk, :], out_sems.at[k]
            )
            cp.start()
            out_copies.append(cp)

        for rdma in sends:
            rdma.wait_send()
        for cp in out_copies:
            cp.wait()

    return pl.pallas_call(
        body,
        out_shape=jax.ShapeDtypeStruct((m, n), jnp.float32),
        in_specs=[
            pl.BlockSpec(memory_space=pltpu.HBM),
            pl.BlockSpec(memory_space=pltpu.VMEM),
            pl.BlockSpec(memory_space=pltpu.VMEM),
        ],
        out_specs=pl.BlockSpec(memory_space=pltpu.HBM),
        scratch_shapes=[
            pltpu.VMEM((m, n), jnp.float32),
            pltpu.VMEM((m, n), jnp.float32),
            pltpu.VMEM((2, m), jnp.float32),
            pltpu.VMEM((3, 2, m), jnp.float32),
            pltpu.SemaphoreType.DMA((NB,)),
            pltpu.SemaphoreType.DMA((NB,)),
            pltpu.SemaphoreType.DMA((3, NB)),
            pltpu.SemaphoreType.DMA((3, NB)),
        ],
        compiler_params=pltpu.CompilerParams(collective_id=0),
    )(x, gamma.reshape(1, n), beta.reshape(1, n))


# device time: 17543 ns/iter; 1.2052x vs baseline; 1.0526x over previous
import jax
import jax.numpy as jnp
from jax import lax
from jax.experimental import pallas as pl
from jax.experimental.pallas import tpu as pltpu

N_DEV = 4
EPS = 1e-5
BLOCKS = (1024, 1024)
NB = len(BLOCKS)
STARTS = tuple(sum(BLOCKS[:i]) for i in range(NB))


def kernel(x, gamma, beta):
    m, n = x.shape
    n_global = n * N_DEV
    assert sum(BLOCKS) == m

    def body(x_hbm, g_ref, b_ref, out_hbm,
             xv_ref, ov_ref, loc_ref, rbuf_ref,
             in_sems, out_sems, send_sems, recv_sems):
        my = lax.axis_index("i")

        in_copies = []
        for k in range(NB):
            cp = pltpu.make_async_copy(
                x_hbm.at[pl.ds(STARTS[k], BLOCKS[k]), :],
                xv_ref.at[pl.ds(STARTS[k], BLOCKS[k]), :],
                in_sems.at[k],
            )
            cp.start()
            in_copies.append(cp)

        barrier_sem = pltpu.get_barrier_semaphore()
        for off in (1, 2, 3):
            pl.semaphore_signal(
                barrier_sem, inc=1,
                device_id=((my + off) % N_DEV,),
                device_id_type=pl.DeviceIdType.MESH,
            )

        ones = jnp.ones((1, n), jnp.float32)
        dims = (((1,), (1,)), ((), ()))

        sends = []
        for k in range(NB):
            blk = pl.ds(STARTS[k], BLOCKS[k])
            in_copies[k].wait()
            xb = xv_ref[STARTS[k]:STARTS[k] + BLOCKS[k], :]
            s = lax.dot_general(ones, xb, dims, preferred_element_type=jnp.float32)
            q = lax.dot_general(ones, xb * xb, dims, preferred_element_type=jnp.float32)
            loc_ref[0:1, STARTS[k]:STARTS[k] + BLOCKS[k]] = s
            loc_ref[1:2, STARTS[k]:STARTS[k] + BLOCKS[k]] = q
            if k == 0:
                pl.semaphore_wait(barrier_sem, 3)
            for off in (2, 1, 3):
                peer = (my + off) % N_DEV
                rdma = pltpu.make_async_remote_copy(
                    src_ref=loc_ref.at[:, blk],
                    dst_ref=rbuf_ref.at[3 - off, :, blk],
                    send_sem=send_sems.at[off - 1, k],
                    recv_sem=recv_sems.at[3 - off, k],
                    device_id=(peer,),
                    device_id_type=pl.DeviceIdType.MESH,
                )
                rdma.start()
                sends.append(rdma)

        out_copies = []
        for k in range(NB):
            blk = pl.ds(STARTS[k], BLOCKS[k])
            for j in range(3):
                recv = pltpu.make_async_remote_copy(
                    src_ref=loc_ref.at[:, blk],
                    dst_ref=rbuf_ref.at[j, :, blk],
                    send_sem=send_sems.at[0, k],
                    recv_sem=recv_sems.at[j, k],
                    device_id=(my,),
                    device_id_type=pl.DeviceIdType.MESH,
                )
                recv.wait_recv()
            sl = slice(STARTS[k], STARTS[k] + BLOCKS[k])
            tot = (loc_ref[:, sl] + rbuf_ref[0, :, sl]
                   + rbuf_ref[1, :, sl] + rbuf_ref[2, :, sl])
            mean_r = tot[0:1, :] / n_global
            var_r = tot[1:2, :] / n_global - mean_r * mean_r
            rstd_r = lax.rsqrt(var_r + EPS)
            mv = jnp.concatenate([mean_r, rstd_r], axis=0)
            t = mv.T
            mean_c = t[:, 0:1]
            rstd_c = t[:, 1:2]
            xb = xv_ref[sl, :]
            ov_ref[sl, :] = (xb - mean_c) * rstd_c * g_ref[:, :] + b_ref[:, :]
            cp = pltpu.make_async_copy(
                ov_ref.at[blk, :], out_hbm.at[blk, :], out_sems.at[k]
            )
            cp.start()
            out_copies.append(cp)

        for rdma in sends:
            rdma.wait_send()
        for cp in out_copies:
            cp.wait()

    return pl.pallas_call(
        body,
        out_shape=jax.ShapeDtypeStruct((m, n), jnp.float32),
        in_specs=[
            pl.BlockSpec(memory_space=pltpu.HBM),
            pl.BlockSpec(memory_space=pltpu.VMEM),
            pl.BlockSpec(memory_space=pltpu.VMEM),
        ],
        out_specs=pl.BlockSpec(memory_space=pltpu.HBM),
        scratch_shapes=[
            pltpu.VMEM((m, n), jnp.float32),
            pltpu.VMEM((m, n), jnp.float32),
            pltpu.VMEM((2, m), jnp.float32),
            pltpu.VMEM((3, 2, m), jnp.float32),
            pltpu.SemaphoreType.DMA((NB,)),
            pltpu.SemaphoreType.DMA((NB,)),
            pltpu.SemaphoreType.DMA((3, NB)),
            pltpu.SemaphoreType.DMA((3, NB)),
        ],
        compiler_params=pltpu.CompilerParams(collective_id=0),
    )(x, gamma.reshape(1, n), beta.reshape(1, n))
